# baseline (speedup 1.0000x reference)
import functools

import jax
import jax.numpy as jnp
from jax import lax
from jax.experimental import pallas as pl
from jax.experimental.pallas import tpu as pltpu, tpu_sc as plsc

N = 100000
E = 3200000
FEAT = 2

CH = 10000
TILES = 16
CHUNKS_PER_TILE = 20
EDGES_PER_TILE = CH * CHUNKS_PER_TILE

NODES_PER_TILE = 6272
N_PAD = NODES_PER_TILE * TILES
NB = 1568
NCHUNKS = NODES_PER_TILE // NB

_mesh = plsc.VectorSubcoreMesh(core_axis_name="c", subcore_axis_name="s")


@functools.partial(
    pl.kernel,
    out_type=jax.ShapeDtypeStruct((2 * 2 * FEAT * N_PAD,), jnp.float32),
    mesh=_mesh,
    scratch_types=[
        pltpu.VMEM_SHARED((N_PAD,), jnp.float32),
        pltpu.VMEM_SHARED((N_PAD,), jnp.float32),
        pltpu.VMEM_SHARED((N_PAD,), jnp.float32),
        pltpu.VMEM_SHARED((N_PAD,), jnp.float32),
        pltpu.VMEM_SHARED((N_PAD,), jnp.float32),
        pltpu.VMEM((CH,), jnp.int32),
        pltpu.VMEM((CH,), jnp.int32),
        pltpu.VMEM((CH,), jnp.int32),
        pltpu.VMEM((CH,), jnp.int32),
        pltpu.VMEM((CH,), jnp.float32),
        pltpu.VMEM((CH,), jnp.float32),
        pltpu.VMEM((CH,), jnp.float32),
        pltpu.VMEM((NB,), jnp.float32),
        pltpu.VMEM((NB,), jnp.float32),
        pltpu.VMEM((NB,), jnp.float32),
        pltpu.VMEM((NB,), jnp.float32),
        pltpu.VMEM((NB,), jnp.float32),
        pltpu.VMEM((NB,), jnp.float32),
        pltpu.VMEM((NB,), jnp.float32),
        pltpu.SemaphoreType.DMA,
        pltpu.SemaphoreType.DMA,
        pltpu.SemaphoreType.DMA,
        pltpu.SemaphoreType.DMA,
        pltpu.SemaphoreType.DMA,
        pltpu.SemaphoreType.DMA,
        pltpu.SemaphoreType.DMA,
        pltpu.SemaphoreType.DMA,
        pltpu.SemaphoreType.DMA,
    ],
)
def _mp_kernel(x0_hbm, x1_hbm, fwd_hbm, rev_hbm, ones_hbm, zeros_hbm,
               out_hbm, table0, table1, acc0, acc1, cnt,
               sidx0, sidx1, didx0, didx1, msg0, msg1, ones_v,
               ns0, ns1, ncnt, nh0, nh1, nr, zbuf,
               sem_si0, sem_si1, sem_di0, sem_di1,
               sem_g0, sem_g1, sem_s0, sem_s1, sem_c):
    c = lax.axis_index("c")
    s = lax.axis_index("s")
    node0 = s * NODES_PER_TILE

    pltpu.sync_copy(ones_hbm, ones_v)
    pltpu.sync_copy(zeros_hbm, zbuf)

    def stage_body(bi, carry):
        bsl = pl.ds(node0 + bi * NB, NB)
        pltpu.sync_copy(x0_hbm.at[bsl], nh0)
        pltpu.sync_copy(nh0, table0.at[bsl])
        pltpu.sync_copy(x1_hbm.at[bsl], nh1)
        pltpu.sync_copy(nh1, table1.at[bsl])
        pltpu.sync_copy(zbuf, acc0.at[bsl])
        pltpu.sync_copy(zbuf, acc1.at[bsl])
        pltpu.sync_copy(zbuf, cnt.at[bsl])
        return carry

    lax.fori_loop(0, NCHUNKS, stage_body, 0)
    plsc.subcore_barrier()

    for r in range(2):
        def edge_phase(edges_hbm, r=r):
            IB = ((sidx0, didx0, sem_si0, sem_di0),
                  (sidx1, didx1, sem_si1, sem_di1))

            def idx_start(ci, b):
                cic = jnp.minimum(ci, CHUNKS_PER_TILE - 1)
                e0 = s * EDGES_PER_TILE + cic * CH
                pltpu.async_copy(edges_hbm.at[pl.ds(e0, CH)], IB[b][0],
                                 IB[b][2])
                pltpu.async_copy(edges_hbm.at[pl.ds(E + e0, CH)], IB[b][1],
                                 IB[b][3])

            def idx_wait(b):
                sl = pl.ds(0, CH)
                pltpu.make_async_copy(edges_hbm.at[sl], IB[b][0],
                                      IB[b][2]).wait()
                pltpu.make_async_copy(edges_hbm.at[sl], IB[b][1],
                                      IB[b][3]).wait()

            idx_start(0, 0)

            def outer(co, carry):
                for b in range(2):
                    ci = 2 * co + b
                    idx_wait(b)
                    idx_start(ci + 1, 1 - b)
                    sidx, didx = IB[b][0], IB[b][1]
                    h_g0 = pltpu.async_copy(table0.at[sidx], msg0, sem_g0)
                    h_g1 = pltpu.async_copy(table1.at[sidx], msg1, sem_g1)
                    if r == 0:
                        h_c = pltpu.async_copy(ones_v, cnt.at[didx], sem_c,
                                               add=True)
                    h_g0.wait()
                    h_s0 = pltpu.async_copy(msg0, acc0.at[didx], sem_s0,
                                            add=True)
                    h_g1.wait()
                    h_s1 = pltpu.async_copy(msg1, acc1.at[didx], sem_s1,
                                            add=True)
                    if r == 0:
                        h_c.wait()
                    h_s0.wait()
                    h_s1.wait()
                return carry

            lax.fori_loop(0, CHUNKS_PER_TILE // 2, outer, 0)
            idx_wait(0)

        @pl.when(c == 0)
        def _():
            edge_phase(fwd_hbm)

        @pl.when(c == 1)
        def _():
            edge_phase(rev_hbm)

        plsc.subcore_barrier()

        def norm_body(bi, carry, r=r):
            b0 = node0 + bi * NB
            bsl = pl.ds(b0, NB)
            pltpu.sync_copy(acc0.at[bsl], ns0)
            pltpu.sync_copy(acc1.at[bsl], ns1)
            pltpu.sync_copy(cnt.at[bsl], ncnt)

            def inner(i, icarry, r=r):
                isl = pl.ds(i * 16, 16)
                cc = ncnt[isl]
                if r == 0:
                    rr = 1.0 / jnp.maximum(cc, 1.0)
                    nr[isl] = rr
                else:
                    rr = cc
                nh0[isl] = ns0[isl] * rr
                nh1[isl] = ns1[isl] * rr
                return icarry

            lax.fori_loop(0, NB // 16, inner, 0)
            if r == 0:
                pltpu.sync_copy(nr, cnt.at[bsl])
                pltpu.sync_copy(zbuf, acc0.at[bsl])
                pltpu.sync_copy(zbuf, acc1.at[bsl])
                pltpu.sync_copy(nh0, table0.at[bsl])
                pltpu.sync_copy(nh1, table1.at[bsl])
            o0 = (c * 4 + 2 * r) * N_PAD + b0
            pltpu.sync_copy(nh0, out_hbm.at[pl.ds(o0, NB)])
            pltpu.sync_copy(nh1, out_hbm.at[pl.ds(o0 + N_PAD, NB)])
            return carry

        lax.fori_loop(0, NCHUNKS, norm_body, 0)
        plsc.subcore_barrier()


@jax.jit
def kernel(topic_entity_one_hot, edge_index, reverse_edge_index):
    x = topic_entity_one_hot.astype(jnp.float32)
    ei = edge_index.astype(jnp.int32)
    rei = reverse_edge_index.astype(jnp.int32)

    fwd = ei.reshape(2 * E)
    rev = rei.reshape(2 * E)

    x0 = jnp.zeros((N_PAD,), jnp.float32).at[:N].set(x[:, 0])
    x1 = jnp.zeros((N_PAD,), jnp.float32).at[:N].set(x[:, 1])
    ones = jnp.ones((CH,), jnp.float32)
    zeros = jnp.zeros((NB,), jnp.float32)

    out = _mp_kernel(x0, x1, fwd, rev, ones, zeros)
    out = out.reshape(2, 2, FEAT, N_PAD)
    return out[:, :, :, :N].transpose(3, 0, 1, 2).reshape(N, 4 * FEAT)


# speedup vs baseline: 1.1861x; 1.1861x over previous
import functools

import jax
import jax.numpy as jnp
from jax import lax
from jax.experimental import pallas as pl
from jax.experimental.pallas import tpu as pltpu, tpu_sc as plsc

N = 100000
E = 3200000
FEAT = 2

CH = 5000
TILES = 16
CHUNKS_PER_TILE = 40
UNROLL = 4
EDGES_PER_TILE = CH * CHUNKS_PER_TILE

NODES_PER_TILE = 6272
N_PAD = NODES_PER_TILE * TILES
NB = 1568
NCHUNKS = NODES_PER_TILE // NB

_mesh = plsc.VectorSubcoreMesh(core_axis_name="c", subcore_axis_name="s")


@functools.partial(
    pl.kernel,
    out_type=jax.ShapeDtypeStruct((2 * 2 * FEAT * N_PAD,), jnp.float32),
    mesh=_mesh,
    scratch_types=[
        pltpu.VMEM_SHARED((N_PAD,), jnp.float32),
        pltpu.VMEM_SHARED((N_PAD,), jnp.float32),
        pltpu.VMEM_SHARED((N_PAD,), jnp.float32),
        pltpu.VMEM_SHARED((N_PAD,), jnp.float32),
        pltpu.VMEM_SHARED((N_PAD,), jnp.float32),
        pltpu.VMEM((CH,), jnp.int32),
        pltpu.VMEM((CH,), jnp.int32),
        pltpu.VMEM((CH,), jnp.int32),
        pltpu.VMEM((CH,), jnp.int32),
        pltpu.VMEM((CH,), jnp.float32),
        pltpu.VMEM((CH,), jnp.float32),
        pltpu.VMEM((CH,), jnp.float32),
        pltpu.VMEM((CH,), jnp.float32),
        pltpu.VMEM((CH,), jnp.float32),
        pltpu.VMEM((NB,), jnp.float32),
        pltpu.VMEM((NB,), jnp.float32),
        pltpu.VMEM((NB,), jnp.float32),
        pltpu.VMEM((NB,), jnp.float32),
        pltpu.VMEM((NB,), jnp.float32),
        pltpu.VMEM((NB,), jnp.float32),
        pltpu.VMEM((NB,), jnp.float32),
        pltpu.SemaphoreType.DMA,
        pltpu.SemaphoreType.DMA,
        pltpu.SemaphoreType.DMA,
        pltpu.SemaphoreType.DMA,
        pltpu.SemaphoreType.DMA,
        pltpu.SemaphoreType.DMA,
        pltpu.SemaphoreType.DMA,
    ],
)
def _mp_kernel(x0_hbm, x1_hbm, fwd_hbm, rev_hbm, ones_hbm, zeros_hbm,
               out_hbm, table0, table1, acc0, acc1, cnt,
               sidx0, sidx1, didx0, didx1, m00, m01, m10, m11, ones_v,
               ns0, ns1, ncnt, nh0, nh1, nr, zbuf,
               sem_si, sem_di, sem_g0, sem_g1, sem_s0, sem_s1, sem_c):
    c = lax.axis_index("c")
    s = lax.axis_index("s")
    node0 = s * NODES_PER_TILE

    pltpu.sync_copy(ones_hbm, ones_v)
    pltpu.sync_copy(zeros_hbm, zbuf)

    def stage_body(bi, carry):
        bsl = pl.ds(node0 + bi * NB, NB)
        pltpu.sync_copy(x0_hbm.at[bsl], nh0)
        pltpu.sync_copy(nh0, table0.at[bsl])
        pltpu.sync_copy(x1_hbm.at[bsl], nh1)
        pltpu.sync_copy(nh1, table1.at[bsl])
        pltpu.sync_copy(zbuf, acc0.at[bsl])
        pltpu.sync_copy(zbuf, acc1.at[bsl])
        pltpu.sync_copy(zbuf, cnt.at[bsl])
        return carry

    lax.fori_loop(0, NCHUNKS, stage_body, 0)
    plsc.subcore_barrier()

    for r in range(2):
        def edge_phase(edges_hbm, r=r):
            BUF = ((sidx0, didx0, m00, m10), (sidx1, didx1, m01, m11))

            def idx_start(ci, b):
                e0 = s * EDGES_PER_TILE + ci * CH
                h0 = pltpu.async_copy(edges_hbm.at[pl.ds(e0, CH)],
                                      BUF[b][0], sem_si)
                h1 = pltpu.async_copy(edges_hbm.at[pl.ds(E + e0, CH)],
                                      BUF[b][1], sem_di)
                return (h0, h1)

            def g_start(b):
                h0 = pltpu.async_copy(table0.at[BUF[b][0]], BUF[b][2], sem_g0)
                h1 = pltpu.async_copy(table1.at[BUF[b][0]], BUF[b][3], sem_g1)
                return (h0, h1)

            def s_start(b):
                h0 = pltpu.async_copy(BUF[b][2], acc0.at[BUF[b][1]], sem_s0,
                                      add=True)
                h1 = pltpu.async_copy(BUF[b][3], acc1.at[BUF[b][1]], sem_s1,
                                      add=True)
                if r == 0:
                    hc = pltpu.async_copy(ones_v, cnt.at[BUF[b][1]], sem_c,
                                          add=True)
                    return (h0, h1, hc)
                return (h0, h1)

            def wait_all(hs):
                for h in hs:
                    h.wait()

            def outer(co, carry):
                c0 = co * UNROLL
                h_idx = idx_start(c0, 0)
                h_s = None
                for j in range(UNROLL):
                    b = j % 2
                    wait_all(h_idx)
                    h_g = g_start(b)
                    if h_s is not None:
                        wait_all(h_s)
                    if j + 1 < UNROLL:
                        h_idx = idx_start(c0 + j + 1, 1 - b)
                    wait_all(h_g)
                    h_s = s_start(b)
                wait_all(h_s)
                return carry

            lax.fori_loop(0, CHUNKS_PER_TILE // UNROLL, outer, 0)

        @pl.when(c == 0)
        def _():
            edge_phase(fwd_hbm)

        @pl.when(c == 1)
        def _():
            edge_phase(rev_hbm)

        plsc.subcore_barrier()

        def norm_body(bi, carry, r=r):
            b0 = node0 + bi * NB
            bsl = pl.ds(b0, NB)
            pltpu.sync_copy(acc0.at[bsl], ns0)
            pltpu.sync_copy(acc1.at[bsl], ns1)
            pltpu.sync_copy(cnt.at[bsl], ncnt)

            def inner(i, icarry, r=r):
                isl = pl.ds(i * 16, 16)
                cc = ncnt[isl]
                if r == 0:
                    rr = 1.0 / jnp.maximum(cc, 1.0)
                    nr[isl] = rr
                else:
                    rr = cc
                nh0[isl] = ns0[isl] * rr
                nh1[isl] = ns1[isl] * rr
                return icarry

            lax.fori_loop(0, NB // 16, inner, 0)
            if r == 0:
                pltpu.sync_copy(nr, cnt.at[bsl])
                pltpu.sync_copy(zbuf, acc0.at[bsl])
                pltpu.sync_copy(zbuf, acc1.at[bsl])
                pltpu.sync_copy(nh0, table0.at[bsl])
                pltpu.sync_copy(nh1, table1.at[bsl])
            o0 = (c * 4 + 2 * r) * N_PAD + b0
            pltpu.sync_copy(nh0, out_hbm.at[pl.ds(o0, NB)])
            pltpu.sync_copy(nh1, out_hbm.at[pl.ds(o0 + N_PAD, NB)])
            return carry

        lax.fori_loop(0, NCHUNKS, norm_body, 0)
        plsc.subcore_barrier()


@jax.jit
def kernel(topic_entity_one_hot, edge_index, reverse_edge_index):
    x = topic_entity_one_hot.astype(jnp.float32)
    ei = edge_index.astype(jnp.int32)
    rei = reverse_edge_index.astype(jnp.int32)

    fwd = ei.reshape(2 * E)
    rev = rei.reshape(2 * E)

    x0 = jnp.zeros((N_PAD,), jnp.float32).at[:N].set(x[:, 0])
    x1 = jnp.zeros((N_PAD,), jnp.float32).at[:N].set(x[:, 1])
    ones = jnp.ones((CH,), jnp.float32)
    zeros = jnp.zeros((NB,), jnp.float32)

    out = _mp_kernel(x0, x1, fwd, rev, ones, zeros)
    out = out.reshape(2, 2, FEAT, N_PAD)
    return out[:, :, :, :N].transpose(3, 0, 1, 2).reshape(N, 4 * FEAT)


# speedup vs baseline: 1.2444x; 1.0491x over previous
import functools

import jax
import jax.numpy as jnp
from jax import lax
from jax.experimental import pallas as pl
from jax.experimental.pallas import tpu as pltpu, tpu_sc as plsc

N = 100000
E = 3200000
FEAT = 2

CH = 10000
TILES = 16
CHUNKS_PER_TILE = 20
EDGES_PER_TILE = CH * CHUNKS_PER_TILE

NODES_PER_TILE = 6272
N_PAD = NODES_PER_TILE * TILES
NB = 3136
NCHUNKS = NODES_PER_TILE // NB

_mesh = plsc.VectorSubcoreMesh(core_axis_name="c", subcore_axis_name="s")


@functools.partial(
    pl.kernel,
    out_type=jax.ShapeDtypeStruct((2 * 2 * FEAT * N_PAD,), jnp.float32),
    mesh=_mesh,
    scratch_types=[
        pltpu.VMEM_SHARED((N_PAD,), jnp.float32),
        pltpu.VMEM_SHARED((N_PAD,), jnp.float32),
        pltpu.VMEM_SHARED((N_PAD,), jnp.float32),
        pltpu.VMEM_SHARED((N_PAD,), jnp.float32),
        pltpu.VMEM_SHARED((N_PAD,), jnp.float32),
        pltpu.VMEM((CH,), jnp.int32),
        pltpu.VMEM((CH,), jnp.int32),
        pltpu.VMEM((CH,), jnp.float32),
        pltpu.VMEM((CH,), jnp.float32),
        pltpu.VMEM((CH,), jnp.float32),
        pltpu.VMEM((NB,), jnp.float32),
        pltpu.VMEM((NB,), jnp.float32),
        pltpu.VMEM((NB,), jnp.float32),
        pltpu.VMEM((NB,), jnp.float32),
        pltpu.VMEM((NB,), jnp.float32),
        pltpu.VMEM((NB,), jnp.float32),
        pltpu.VMEM((NB,), jnp.float32),
        pltpu.SemaphoreType.DMA,
        pltpu.SemaphoreType.DMA,
        pltpu.SemaphoreType.DMA,
        pltpu.SemaphoreType.DMA,
        pltpu.SemaphoreType.DMA,
        pltpu.SemaphoreType.DMA,
        pltpu.SemaphoreType.DMA,
    ],
)
def _mp_kernel(x0_hbm, x1_hbm, fwd_hbm, rev_hbm, ones_hbm, zeros_hbm,
               out_hbm, table0, table1, acc0, acc1, cnt,
               sidx, didx, msg0, msg1, ones_v,
               ns0, ns1, ncnt, nh0, nh1, nr, zbuf,
               sem_si, sem_di, sem_g0, sem_g1, sem_s0, sem_s1, sem_c):
    c = lax.axis_index("c")
    s = lax.axis_index("s")
    node0 = s * NODES_PER_TILE

    pltpu.sync_copy(ones_hbm, ones_v)
    pltpu.sync_copy(zeros_hbm, zbuf)

    def stage_body(bi, carry):
        bsl = pl.ds(node0 + bi * NB, NB)
        pltpu.sync_copy(x0_hbm.at[bsl], nh0)
        pltpu.sync_copy(nh0, table0.at[bsl])
        pltpu.sync_copy(x1_hbm.at[bsl], nh1)
        pltpu.sync_copy(nh1, table1.at[bsl])
        pltpu.sync_copy(zbuf, acc0.at[bsl])
        pltpu.sync_copy(zbuf, acc1.at[bsl])
        pltpu.sync_copy(zbuf, cnt.at[bsl])
        return carry

    lax.fori_loop(0, NCHUNKS, stage_body, 0)
    plsc.subcore_barrier()

    for r in range(2):
        def edge_phase(edges_hbm, r=r):
            def chunk_body(ci, carry):
                e0 = s * EDGES_PER_TILE + ci * CH
                h_si = pltpu.async_copy(
                    edges_hbm.at[pl.ds(e0, CH)], sidx, sem_si)
                h_di = pltpu.async_copy(
                    edges_hbm.at[pl.ds(E + e0, CH)], didx, sem_di)
                h_si.wait()
                h_g0 = pltpu.async_copy(table0.at[sidx], msg0, sem_g0)
                h_g1 = pltpu.async_copy(table1.at[sidx], msg1, sem_g1)
                h_di.wait()
                if r == 0:
                    h_c = pltpu.async_copy(ones_v, cnt.at[didx], sem_c,
                                           add=True)
                h_g0.wait()
                h_s0 = pltpu.async_copy(msg0, acc0.at[didx], sem_s0, add=True)
                h_g1.wait()
                h_s1 = pltpu.async_copy(msg1, acc1.at[didx], sem_s1, add=True)
                if r == 0:
                    h_c.wait()
                h_s0.wait()
                h_s1.wait()
                return carry

            lax.fori_loop(0, CHUNKS_PER_TILE, chunk_body, 0)

        @pl.when(c == 0)
        def _():
            edge_phase(fwd_hbm)

        @pl.when(c == 1)
        def _():
            edge_phase(rev_hbm)

        plsc.subcore_barrier()

        def norm_body(bi, carry, r=r):
            b0 = node0 + bi * NB
            bsl = pl.ds(b0, NB)
            pltpu.sync_copy(acc0.at[bsl], ns0)
            pltpu.sync_copy(acc1.at[bsl], ns1)
            pltpu.sync_copy(cnt.at[bsl], ncnt)

            def inner(i, icarry, r=r):
                isl = pl.ds(i * 16, 16)
                cc = ncnt[isl]
                if r == 0:
                    rr = 1.0 / jnp.maximum(cc, 1.0)
                    nr[isl] = rr
                else:
                    rr = cc
                nh0[isl] = ns0[isl] * rr
                nh1[isl] = ns1[isl] * rr
                return icarry

            lax.fori_loop(0, NB // 16, inner, 0)
            if r == 0:
                pltpu.sync_copy(nr, cnt.at[bsl])
                pltpu.sync_copy(zbuf, acc0.at[bsl])
                pltpu.sync_copy(zbuf, acc1.at[bsl])
                pltpu.sync_copy(nh0, table0.at[bsl])
                pltpu.sync_copy(nh1, table1.at[bsl])
            o0 = (c * 4 + 2 * r) * N_PAD + b0
            pltpu.sync_copy(nh0, out_hbm.at[pl.ds(o0, NB)])
            pltpu.sync_copy(nh1, out_hbm.at[pl.ds(o0 + N_PAD, NB)])
            return carry

        lax.fori_loop(0, NCHUNKS, norm_body, 0)
        plsc.subcore_barrier()


@jax.jit
def kernel(topic_entity_one_hot, edge_index, reverse_edge_index):
    x = topic_entity_one_hot.astype(jnp.float32)
    ei = edge_index.astype(jnp.int32)
    rei = reverse_edge_index.astype(jnp.int32)

    fwd = ei.reshape(2 * E)
    rev = rei.reshape(2 * E)

    x0 = jnp.zeros((N_PAD,), jnp.float32).at[:N].set(x[:, 0])
    x1 = jnp.zeros((N_PAD,), jnp.float32).at[:N].set(x[:, 1])
    ones = jnp.ones((CH,), jnp.float32)
    zeros = jnp.zeros((NB,), jnp.float32)

    out = _mp_kernel(x0, x1, fwd, rev, ones, zeros)
    out = out.reshape(2, 2, FEAT, N_PAD)
    return out[:, :, :, :N].transpose(3, 0, 1, 2).reshape(N, 4 * FEAT)


# speedup vs baseline: 1.2517x; 1.0059x over previous
import functools

import jax
import jax.numpy as jnp
from jax import lax
from jax.experimental import pallas as pl
from jax.experimental.pallas import tpu as pltpu, tpu_sc as plsc

N = 100000
E = 3200000
FEAT = 2

CH = 10000
TILES = 16
CHUNKS_PER_TILE = 20
EDGES_PER_TILE = CH * CHUNKS_PER_TILE

NODES_PER_TILE = 6272
N_PAD = NODES_PER_TILE * TILES
NB = 3136
NCHUNKS = NODES_PER_TILE // NB

_mesh = plsc.VectorSubcoreMesh(core_axis_name="c", subcore_axis_name="s")


@functools.partial(
    pl.kernel,
    out_type=jax.ShapeDtypeStruct((2 * 2 * FEAT * N_PAD,), jnp.float32),
    mesh=_mesh,
    scratch_types=[
        pltpu.VMEM_SHARED((N_PAD,), jnp.float32),
        pltpu.VMEM_SHARED((N_PAD,), jnp.float32),
        pltpu.VMEM_SHARED((N_PAD,), jnp.float32),
        pltpu.VMEM_SHARED((N_PAD,), jnp.float32),
        pltpu.VMEM_SHARED((N_PAD,), jnp.float32),
        pltpu.VMEM((CH,), jnp.int32),
        pltpu.VMEM((CH,), jnp.int32),
        pltpu.VMEM((CH,), jnp.float32),
        pltpu.VMEM((CH,), jnp.float32),
        pltpu.VMEM((CH,), jnp.float32),
        pltpu.VMEM((NB,), jnp.float32),
        pltpu.VMEM((NB,), jnp.float32),
        pltpu.VMEM((NB,), jnp.float32),
        pltpu.VMEM((NB,), jnp.float32),
        pltpu.VMEM((NB,), jnp.float32),
        pltpu.VMEM((NB,), jnp.float32),
        pltpu.VMEM((NB,), jnp.float32),
        pltpu.SemaphoreType.DMA,
        pltpu.SemaphoreType.DMA,
        pltpu.SemaphoreType.DMA,
        pltpu.SemaphoreType.DMA,
        pltpu.SemaphoreType.DMA,
        pltpu.SemaphoreType.DMA,
        pltpu.SemaphoreType.DMA,
    ],
)
def _mp_kernel(x0_hbm, x1_hbm, fwd_hbm, rev_hbm, ones_hbm, zeros_hbm,
               out_hbm, table0, table1, acc0, acc1, cnt,
               sidx, didx, msg0, msg1, ones_v,
               ns0, ns1, ncnt, nh0, nh1, nr, zbuf,
               sem_si, sem_di, sem_g0, sem_g1, sem_s0, sem_s1, sem_c):
    c = lax.axis_index("c")
    s = lax.axis_index("s")
    node0 = s * NODES_PER_TILE

    pltpu.sync_copy(ones_hbm, ones_v)
    pltpu.sync_copy(zeros_hbm, zbuf)

    def stage_body(bi, carry):
        bsl = pl.ds(node0 + bi * NB, NB)
        h_x0 = pltpu.async_copy(x0_hbm.at[bsl], nh0, sem_si)
        h_x1 = pltpu.async_copy(x1_hbm.at[bsl], nh1, sem_di)
        h_a0 = pltpu.async_copy(zbuf, acc0.at[bsl], sem_g0)
        h_a1 = pltpu.async_copy(zbuf, acc1.at[bsl], sem_g1)
        h_cn = pltpu.async_copy(zbuf, cnt.at[bsl], sem_c)
        h_x0.wait()
        h_t0 = pltpu.async_copy(nh0, table0.at[bsl], sem_s0)
        h_x1.wait()
        h_t1 = pltpu.async_copy(nh1, table1.at[bsl], sem_s1)
        h_a0.wait()
        h_a1.wait()
        h_cn.wait()
        h_t0.wait()
        h_t1.wait()
        return carry

    lax.fori_loop(0, NCHUNKS, stage_body, 0)
    plsc.subcore_barrier()

    for r in range(2):
        def edge_phase(edges_hbm, r=r):
            def chunk_body(ci, carry):
                e0 = s * EDGES_PER_TILE + ci * CH
                h_si = pltpu.async_copy(
                    edges_hbm.at[pl.ds(e0, CH)], sidx, sem_si)
                h_di = pltpu.async_copy(
                    edges_hbm.at[pl.ds(E + e0, CH)], didx, sem_di)
                h_si.wait()
                h_g0 = pltpu.async_copy(table0.at[sidx], msg0, sem_g0)
                h_g1 = pltpu.async_copy(table1.at[sidx], msg1, sem_g1)
                h_di.wait()
                if r == 0:
                    h_c = pltpu.async_copy(ones_v, cnt.at[didx], sem_c,
                                           add=True)
                h_g0.wait()
                h_s0 = pltpu.async_copy(msg0, acc0.at[didx], sem_s0, add=True)
                h_g1.wait()
                h_s1 = pltpu.async_copy(msg1, acc1.at[didx], sem_s1, add=True)
                if r == 0:
                    h_c.wait()
                h_s0.wait()
                h_s1.wait()
                return carry

            lax.fori_loop(0, CHUNKS_PER_TILE, chunk_body, 0)

        @pl.when(c == 0)
        def _():
            edge_phase(fwd_hbm)

        @pl.when(c == 1)
        def _():
            edge_phase(rev_hbm)

        plsc.subcore_barrier()

        def norm_body(bi, carry, r=r):
            b0 = node0 + bi * NB
            bsl = pl.ds(b0, NB)
            h_l0 = pltpu.async_copy(acc0.at[bsl], ns0, sem_si)
            h_l1 = pltpu.async_copy(acc1.at[bsl], ns1, sem_di)
            h_lc = pltpu.async_copy(cnt.at[bsl], ncnt, sem_c)
            h_l0.wait()
            h_l1.wait()
            h_lc.wait()

            def inner(i, icarry, r=r):
                isl = pl.ds(i * 16, 16)
                cc = ncnt[isl]
                if r == 0:
                    rr = 1.0 / jnp.maximum(cc, 1.0)
                    nr[isl] = rr
                else:
                    rr = cc
                nh0[isl] = ns0[isl] * rr
                nh1[isl] = ns1[isl] * rr
                return icarry

            lax.fori_loop(0, NB // 16, inner, 0)
            hs = []
            if r == 0:
                hs.append(pltpu.async_copy(nr, cnt.at[bsl], sem_c))
                hs.append(pltpu.async_copy(zbuf, acc0.at[bsl], sem_si))
                hs.append(pltpu.async_copy(zbuf, acc1.at[bsl], sem_di))
                hs.append(pltpu.async_copy(nh0, table0.at[bsl], sem_g0))
                hs.append(pltpu.async_copy(nh1, table1.at[bsl], sem_g1))
            o0 = (c * 4 + 2 * r) * N_PAD + b0
            hs.append(pltpu.async_copy(nh0, out_hbm.at[pl.ds(o0, NB)],
                                       sem_s0))
            hs.append(pltpu.async_copy(nh1, out_hbm.at[pl.ds(o0 + N_PAD, NB)],
                                       sem_s1))
            for h in hs:
                h.wait()
            return carry

        lax.fori_loop(0, NCHUNKS, norm_body, 0)
        plsc.subcore_barrier()


@jax.jit
def kernel(topic_entity_one_hot, edge_index, reverse_edge_index):
    x = topic_entity_one_hot.astype(jnp.float32)
    ei = edge_index.astype(jnp.int32)
    rei = reverse_edge_index.astype(jnp.int32)

    fwd = ei.reshape(2 * E)
    rev = rei.reshape(2 * E)

    x0 = jnp.zeros((N_PAD,), jnp.float32).at[:N].set(x[:, 0])
    x1 = jnp.zeros((N_PAD,), jnp.float32).at[:N].set(x[:, 1])
    ones = jnp.ones((CH,), jnp.float32)
    zeros = jnp.zeros((NB,), jnp.float32)

    out = _mp_kernel(x0, x1, fwd, rev, ones, zeros)
    out = out.reshape(2, 2, FEAT, N_PAD)
    return out[:, :, :, :N].transpose(3, 0, 1, 2).reshape(N, 4 * FEAT)


# speedup vs baseline: 1.2538x; 1.0017x over previous
import functools

import jax
import jax.numpy as jnp
from jax import lax
from jax.experimental import pallas as pl
from jax.experimental.pallas import tpu as pltpu, tpu_sc as plsc

N = 100000
E = 3200000
FEAT = 2

CH = 10000
TILES = 16
CHUNKS_PER_TILE = 20
EDGES_PER_TILE = CH * CHUNKS_PER_TILE

NODES_PER_TILE = 6272
N_PAD = NODES_PER_TILE * TILES
NB = 3136
NCHUNKS = NODES_PER_TILE // NB

_mesh = plsc.VectorSubcoreMesh(core_axis_name="c", subcore_axis_name="s")


@functools.partial(
    pl.kernel,
    out_type=jax.ShapeDtypeStruct((2 * 2 * FEAT * N_PAD,), jnp.float32),
    mesh=_mesh,
    scratch_types=[
        pltpu.VMEM_SHARED((N_PAD,), jnp.float32),
        pltpu.VMEM_SHARED((N_PAD,), jnp.float32),
        pltpu.VMEM_SHARED((N_PAD,), jnp.float32),
        pltpu.VMEM_SHARED((N_PAD,), jnp.float32),
        pltpu.VMEM_SHARED((N_PAD,), jnp.float32),
        pltpu.VMEM((CH,), jnp.int32),
        pltpu.VMEM((CH,), jnp.int32),
        pltpu.VMEM((CH,), jnp.float32),
        pltpu.VMEM((CH,), jnp.float32),
        pltpu.VMEM((CH,), jnp.float32),
        pltpu.VMEM((NB,), jnp.float32),
        pltpu.VMEM((NB,), jnp.float32),
        pltpu.VMEM((NB,), jnp.float32),
        pltpu.VMEM((NB,), jnp.float32),
        pltpu.VMEM((NB,), jnp.float32),
        pltpu.VMEM((NB,), jnp.float32),
        pltpu.VMEM((NB,), jnp.float32),
        pltpu.SemaphoreType.DMA,
        pltpu.SemaphoreType.DMA,
        pltpu.SemaphoreType.DMA,
        pltpu.SemaphoreType.DMA,
        pltpu.SemaphoreType.DMA,
        pltpu.SemaphoreType.DMA,
        pltpu.SemaphoreType.DMA,
    ],
)
def _mp_kernel(x0_hbm, x1_hbm, fwd_hbm, rev_hbm, ones_hbm, zeros_hbm,
               out_hbm, table0, table1, acc0, acc1, cnt,
               sidx, didx, msg0, msg1, ones_v,
               ns0, ns1, ncnt, nh0, nh1, nr, zbuf,
               sem_si, sem_di, sem_g0, sem_g1, sem_s0, sem_s1, sem_c):
    c = lax.axis_index("c")
    s = lax.axis_index("s")
    node0 = s * NODES_PER_TILE

    pltpu.sync_copy(ones_hbm, ones_v)
    pltpu.sync_copy(zeros_hbm, zbuf)

    def stage_body(bi, carry):
        bsl = pl.ds(node0 + bi * NB, NB)
        h_x0 = pltpu.async_copy(x0_hbm.at[bsl], nh0, sem_si)
        h_x1 = pltpu.async_copy(x1_hbm.at[bsl], nh1, sem_di)
        h_a0 = pltpu.async_copy(zbuf, acc0.at[bsl], sem_g0)
        h_a1 = pltpu.async_copy(zbuf, acc1.at[bsl], sem_g1)
        h_cn = pltpu.async_copy(zbuf, cnt.at[bsl], sem_c)
        h_x0.wait()
        h_t0 = pltpu.async_copy(nh0, table0.at[bsl], sem_s0)
        h_x1.wait()
        h_t1 = pltpu.async_copy(nh1, table1.at[bsl], sem_s1)
        h_a0.wait()
        h_a1.wait()
        h_cn.wait()
        h_t0.wait()
        h_t1.wait()
        return carry

    lax.fori_loop(0, NCHUNKS, stage_body, 0)
    plsc.subcore_barrier()

    for r in range(2):
        def edge_phase(edges_hbm, r=r):
            def chunk_body(ci, carry):
                e0 = s * EDGES_PER_TILE + ci * CH
                h_si = pltpu.async_copy(
                    edges_hbm.at[pl.ds(e0, CH)], sidx, sem_si)
                h_di = pltpu.async_copy(
                    edges_hbm.at[pl.ds(E + e0, CH)], didx, sem_di)
                h_si.wait()
                h_g0 = pltpu.async_copy(table0.at[sidx], msg0, sem_g0)
                h_g1 = pltpu.async_copy(table1.at[sidx], msg1, sem_g1)
                h_di.wait()
                if r == 0:
                    h_c = pltpu.async_copy(ones_v, cnt.at[didx], sem_c,
                                           add=True)
                h_g0.wait()
                h_s0 = pltpu.async_copy(msg0, acc0.at[didx], sem_s0, add=True)
                h_g1.wait()
                h_s1 = pltpu.async_copy(msg1, acc1.at[didx], sem_s1, add=True)
                if r == 0:
                    h_c.wait()
                h_s0.wait()
                h_s1.wait()
                return carry

            lax.fori_loop(0, CHUNKS_PER_TILE, chunk_body, 0)

        @pl.when(c == 0)
        def _():
            edge_phase(fwd_hbm)

        @pl.when(c == 1)
        def _():
            edge_phase(rev_hbm)

        plsc.subcore_barrier()

        def norm_body(bi, carry, r=r):
            b0 = node0 + bi * NB
            bsl = pl.ds(b0, NB)
            h_l0 = pltpu.async_copy(acc0.at[bsl], ns0, sem_si)
            h_l1 = pltpu.async_copy(acc1.at[bsl], ns1, sem_di)
            h_lc = pltpu.async_copy(cnt.at[bsl], ncnt, sem_c)
            h_l0.wait()
            h_l1.wait()
            h_lc.wait()

            def inner(i, icarry, r=r):
                isl = pl.ds(i * 16, 16)
                cc = ncnt[isl]
                if r == 0:
                    rr = 1.0 / jnp.maximum(cc, 1.0)
                    nr[isl] = rr
                else:
                    rr = cc
                nh0[isl] = ns0[isl] * rr
                nh1[isl] = ns1[isl] * rr
                return icarry

            lax.fori_loop(0, NB // 16, inner, 0)
            hs = []
            if r == 0:
                hs.append(pltpu.async_copy(nr, cnt.at[bsl], sem_c))
                hs.append(pltpu.async_copy(zbuf, acc0.at[bsl], sem_si))
                hs.append(pltpu.async_copy(zbuf, acc1.at[bsl], sem_di))
                hs.append(pltpu.async_copy(nh0, table0.at[bsl], sem_g0))
                hs.append(pltpu.async_copy(nh1, table1.at[bsl], sem_g1))
            o0 = (c * 4 + 2 * r) * N_PAD + b0
            hs.append(pltpu.async_copy(nh0, out_hbm.at[pl.ds(o0, NB)],
                                       sem_s0))
            hs.append(pltpu.async_copy(nh1, out_hbm.at[pl.ds(o0 + N_PAD, NB)],
                                       sem_s1))
            for h in hs:
                h.wait()
            return carry

        lax.fori_loop(0, NCHUNKS, norm_body, 0)
        plsc.subcore_barrier()


@jax.jit
def kernel(topic_entity_one_hot, edge_index, reverse_edge_index):
    x = topic_entity_one_hot.astype(jnp.float32)
    ei = edge_index.astype(jnp.int32)
    rei = reverse_edge_index.astype(jnp.int32)

    fwd = ei.reshape(2 * E)
    rev = rei.reshape(2 * E)

    m0 = jnp.array([[1.0, 0.0]], jnp.float32)
    m1 = jnp.array([[0.0, 1.0]], jnp.float32)
    x0 = jnp.zeros((N_PAD,), jnp.float32).at[:N].set((x * m0).sum(axis=1))
    x1 = jnp.zeros((N_PAD,), jnp.float32).at[:N].set((x * m1).sum(axis=1))
    ones = jnp.ones((CH,), jnp.float32)
    zeros = jnp.zeros((NB,), jnp.float32)

    out = _mp_kernel(x0, x1, fwd, rev, ones, zeros)
    out = out.reshape(2, 2, FEAT, N_PAD)
    return out[:, :, :, :N].transpose(3, 0, 1, 2).reshape(N, 4 * FEAT)
